# baseline (speedup 1.0000x reference)
import jax
import jax.numpy as jnp
from jax.experimental import pallas as pl

_B, _C, _H, _W = 16, 1, 512, 512
_TOPK = 180
_TOL = 3
_IDX_BITS = 18
_NJ = 4
_S = 384
_CID_BITS = 11
_MININT = -2147483648


def _monotone_key(x):
    b = jax.lax.bitcast_convert_type(x, jnp.int32)
    return jnp.where(b < 0, b ^ jnp.int32(0x7FFFFFFF), b)


def _kth3(arrs, ks):
    minint = jnp.int32(_MININT)
    one = jnp.int32(1)

    def body(i, ubs):
        bit = 31 - i
        out = []
        for u, a, k in zip(ubs, arrs, ks):
            p = u | jnp.left_shift(one, bit)
            cnt = jnp.sum((a >= (p ^ minint)).astype(jnp.int32))
            out.append(jnp.where(cnt >= k, p, u))
        return tuple(out)

    ubs = jax.lax.fori_loop(0, 32, body, (jnp.int32(0),) * 3)
    return tuple(u ^ minint for u in ubs)


def _cut3(ties, ids, rs, nbits):
    one = jnp.int32(1)

    def body(i, ubs):
        bit = (nbits - 1) - i
        out = []
        for u, tie, idv, r in zip(ubs, ties, ids, rs):
            s = u | jnp.left_shift(one, bit)
            cnt = jnp.sum((tie & (idv < s)).astype(jnp.int32))
            out.append(jnp.where(cnt < r, s, u))
        return tuple(out)

    ubs = jax.lax.fori_loop(0, nbits, body, (jnp.int32(0),) * 3)
    return ubs


def _chunk_maxes(keys):
    cms = [
        jnp.max(keys[:, j * 128 : (j + 1) * 128], axis=1, keepdims=True)
        for j in range(_NJ)
    ]
    return jnp.concatenate(cms, axis=1)


def _compact(x, selc, pos):
    h, w = x.shape
    nc = h * _NJ
    lane_r = jax.lax.broadcasted_iota(jnp.int32, (nc, _S), 1)
    hcol = jax.lax.broadcasted_iota(jnp.int32, (h, 1), 0)
    h_lo = (hcol & 255).astype(jnp.float32)
    h_hi = jax.lax.shift_right_logical(hcol, 8).astype(jnp.float32)

    pos_cat = jnp.concatenate(
        [pos[:, j : j + 1] for j in range(_NJ)], axis=0
    ).astype(jnp.int32)
    sel_cat = jnp.concatenate([selc[:, j : j + 1] for j in range(_NJ)], axis=0)
    q = ((lane_r == pos_cat) & (sel_cat > 0.5)).astype(jnp.float32)

    rhs = jnp.concatenate(
        [jnp.concatenate([x[:, j * 128 : (j + 1) * 128] for j in range(_NJ)], axis=0)]
        + [
            jnp.concatenate([h_lo for _ in range(_NJ)], axis=0),
            jnp.concatenate([h_hi for _ in range(_NJ)], axis=0),
            jnp.concatenate(
                [jnp.full((h, 1), float(j), jnp.float32) for j in range(_NJ)], axis=0
            ),
        ],
        axis=1,
    )

    g = jax.lax.dot_general(
        q, rhs, (((0,), (0,)), ((), ())), precision=jax.lax.Precision.HIGHEST
    )
    cand_keys = _monotone_key(g[:, :128])
    h_r = (g[:, 129:130] * 256.0 + g[:, 128:129]).astype(jnp.int32)
    base = h_r * w + g[:, 130:131].astype(jnp.int32) * 128
    cand_idx = base + jax.lax.broadcasted_iota(jnp.int32, (_S, 128), 1)
    return cand_keys, cand_idx


def _positions(selc, tril):
    s = selc
    rowtot = s[:, 0:1] + s[:, 1:2] + s[:, 2:3] + s[:, 3:4]
    cumex = jax.lax.dot_general(
        tril, rowtot, (((1,), (0,)), ((), ()))
    )
    p0 = cumex
    p1 = p0 + s[:, 0:1]
    p2 = p1 + s[:, 1:2]
    p3 = p2 + s[:, 2:3]
    return jnp.concatenate([p0, p1, p2, p3], axis=1)


def _dilate(m):
    h, w = m.shape
    f = m
    for d in (1, 2, 3):
        up = jnp.concatenate([m[d:, :], jnp.zeros((d, w), jnp.float32)], axis=0)
        dnn = jnp.concatenate([jnp.zeros((d, w), jnp.float32), m[: h - d, :]], axis=0)
        f = jnp.maximum(f, jnp.maximum(up, dnn))
    g = f
    for d in (1, 2, 3):
        lf = jnp.concatenate([f[:, d:], jnp.zeros((h, d), jnp.float32)], axis=1)
        rt = jnp.concatenate([jnp.zeros((h, d), jnp.float32), f[:, : w - d]], axis=1)
        g = jnp.maximum(g, jnp.maximum(lf, rt))
    return g


def _body(rv_ref, ri_ref, rf_ref, out_ref):
    pid = pl.program_id(0)
    h, w = rv_ref.shape[1], rv_ref.shape[2]
    row = jax.lax.broadcasted_iota(jnp.int32, (h, w), 0)
    col = jax.lax.broadcasted_iota(jnp.int32, (h, w), 1)
    idx = row * w + col
    tril = (
        jax.lax.broadcasted_iota(jnp.int32, (h, h), 0)
        > jax.lax.broadcasted_iota(jnp.int32, (h, h), 1)
    ).astype(jnp.float32)
    cid = (
        jax.lax.broadcasted_iota(jnp.int32, (h, _NJ), 0) * _NJ
        + jax.lax.broadcasted_iota(jnp.int32, (h, _NJ), 1)
    )

    xs = [r[0] for r in (rv_ref, ri_ref, rf_ref)]
    keys = [_monotone_key(x) for x in xs]
    ckeys = [_chunk_maxes(kk) for kk in keys]

    cts = _kth3(ckeys, (_S, _S, _S))
    ctie = [ck == t for ck, t in zip(ckeys, cts)]
    crs = [
        jnp.int32(_S) - jnp.sum((ck > t).astype(jnp.int32))
        for ck, t in zip(ckeys, cts)
    ]
    ccuts = _cut3(ctie, (cid, cid, cid), crs, _CID_BITS)
    selcs = [
        ((ck > t) | (ti & (cid <= cu))).astype(jnp.float32)
        for ck, t, ti, cu in zip(ckeys, cts, ctie, ccuts)
    ]

    cands = [
        _compact(x, sc, _positions(sc, tril)) for x, sc in zip(xs, selcs)
    ]
    ckq = [ckv for ckv, _ in cands]
    cix = [civ for _, civ in cands]
    kks = (_TOPK, _TOPK, 2 * _TOPK)
    tss = _kth3(ckq, kks)
    ties = [cq == t for cq, t in zip(ckq, tss)]
    rrs = [
        jnp.int32(k) - jnp.sum((cq > t).astype(jnp.int32))
        for cq, t, k in zip(ckq, tss, kks)
    ]
    cuts = _cut3(ties, cix, rrs, _IDX_BITS)

    masks = [
        ((kk > t) | ((kk == t) & (idx <= cu))).astype(jnp.float32)
        for kk, t, cu in zip(keys, tss, cuts)
    ]

    src = jnp.maximum(masks[0], masks[1])
    dil = _dilate(masks[2])
    cover = jnp.sum(src * dil)
    denom = jnp.maximum(jnp.sum(src), 1.0)
    contrib = (1.0 - cover / denom) * jnp.float32(1.0 / _B)

    @pl.when(pid == 0)
    def _():
        out_ref[...] = jnp.zeros_like(out_ref)

    out_ref[...] += contrib


def kernel(rv, ri, rf):
    b, c, h, w = rv.shape
    rv3 = rv.reshape(b, c * h, w)
    ri3 = ri.reshape(b, c * h, w)
    rf3 = rf.reshape(b, c * h, w)
    spec = pl.BlockSpec((1, c * h, w), lambda i: (i, 0, 0))
    out = pl.pallas_call(
        _body,
        grid=(b,),
        in_specs=[spec, spec, spec],
        out_specs=pl.BlockSpec((1, 128), lambda i: (0, 0)),
        out_shape=jax.ShapeDtypeStruct((1, 128), jnp.float32),
    )(rv3, ri3, rf3)
    return out[0, 0]


# speedup vs baseline: 1.1459x; 1.1459x over previous
import jax
import jax.numpy as jnp
from jax.experimental import pallas as pl

_B, _C, _H, _W = 16, 1, 512, 512
_TOPK = 180
_TOL = 3
_IDX_BITS = 18
_NJ = 4
_S = 384
_CID_BITS = 12
_MININT = -2147483648


def _monotone_key(x):
    b = jax.lax.bitcast_convert_type(x, jnp.int32)
    return jnp.where(b < 0, b ^ jnp.int32(0x7FFFFFFF), b)


def _csum(pred):
    return jnp.sum(pred.astype(jnp.int32), keepdims=True)


def _kth3(arrs, ks):
    minint = jnp.int32(_MININT)

    def body(i, ubs):
        shift = 30 - 2 * i
        out = []
        for u, a, k in zip(ubs, arrs, ks):
            p1 = u | jnp.left_shift(jnp.int32(1), shift)
            p2 = u | jnp.left_shift(jnp.int32(2), shift)
            p3 = u | jnp.left_shift(jnp.int32(3), shift)
            c1 = _csum(a >= (p1 ^ minint))
            c2 = _csum(a >= (p2 ^ minint))
            c3 = _csum(a >= (p3 ^ minint))
            nu = jnp.where(c3 >= k, p3, jnp.where(c2 >= k, p2, jnp.where(c1 >= k, p1, u)))
            out.append(nu)
        return tuple(out)

    init = (jnp.zeros((1, 1), jnp.int32),) * 3
    ubs = jax.lax.fori_loop(0, 16, body, init)
    return tuple(u ^ minint for u in ubs)


def _cut3(ties, ids, rs, nbits):
    def body(i, ubs):
        shift = (nbits - 2) - 2 * i
        out = []
        for u, tie, idv, r in zip(ubs, ties, ids, rs):
            s1 = u | jnp.left_shift(jnp.int32(1), shift)
            s2 = u | jnp.left_shift(jnp.int32(2), shift)
            s3 = u | jnp.left_shift(jnp.int32(3), shift)
            c1 = _csum(tie & (idv < s1))
            c2 = _csum(tie & (idv < s2))
            c3 = _csum(tie & (idv < s3))
            nu = jnp.where(c3 < r, s3, jnp.where(c2 < r, s2, jnp.where(c1 < r, s1, u)))
            out.append(nu)
        return tuple(out)

    init = (jnp.zeros((1, 1), jnp.int32),) * 3
    return jax.lax.fori_loop(0, nbits // 2, body, init)


def _chunk_maxes(keys):
    cms = [
        jnp.max(keys[:, j * 128 : (j + 1) * 128], axis=1, keepdims=True)
        for j in range(_NJ)
    ]
    return jnp.concatenate(cms, axis=1)


def _positions(selc, tril):
    s = selc
    rowtot = s[:, 0:1] + s[:, 1:2] + s[:, 2:3] + s[:, 3:4]
    cumex = jax.lax.dot_general(
        tril, rowtot, (((1,), (0,)), ((), ()))
    )
    p0 = cumex
    p1 = p0 + s[:, 0:1]
    p2 = p1 + s[:, 1:2]
    p3 = p2 + s[:, 2:3]
    return jnp.concatenate([p0, p1, p2, p3], axis=1)


def _compact(keys, selc, pos):
    minint = jnp.int32(_MININT)
    h, w = keys.shape
    nc = h * _NJ
    ubk = keys ^ minint
    lane_r = jax.lax.broadcasted_iota(jnp.int32, (nc, _S), 1)
    hcol = jax.lax.broadcasted_iota(jnp.int32, (h, 1), 0)
    h_lo = (hcol & 255).astype(jnp.float32)
    h_hi = jax.lax.shift_right_logical(hcol, 8).astype(jnp.float32)

    pos_cat = jnp.concatenate(
        [pos[:, j : j + 1] for j in range(_NJ)], axis=0
    ).astype(jnp.int32)
    sel_cat = jnp.concatenate([selc[:, j : j + 1] for j in range(_NJ)], axis=0)
    q = ((lane_r == pos_cat) & (sel_cat > 0.5)).astype(jnp.float32)

    dn = (((0,), (0,)), ((), ()))
    gp = []
    for p in range(4):
        plane = jnp.concatenate(
            [
                (
                    jax.lax.shift_right_logical(
                        ubk[:, j * 128 : (j + 1) * 128], 8 * (3 - p)
                    )
                    & 255
                ).astype(jnp.float32)
                for j in range(_NJ)
            ],
            axis=0,
        )
        if p == 3:
            plane = jnp.concatenate(
                [
                    plane,
                    jnp.concatenate([h_lo for _ in range(_NJ)], axis=0),
                    jnp.concatenate([h_hi for _ in range(_NJ)], axis=0),
                    jnp.concatenate(
                        [jnp.full((h, 1), float(j), jnp.float32) for j in range(_NJ)],
                        axis=0,
                    ),
                ],
                axis=1,
            )
        gp.append(jax.lax.dot_general(q, plane, dn))

    ip = [g.astype(jnp.int32) for g in (gp[0], gp[1], gp[2], gp[3][:, :128])]
    cand_keys = (
        jnp.left_shift(ip[0], 24)
        | jnp.left_shift(ip[1], 16)
        | jnp.left_shift(ip[2], 8)
        | ip[3]
    ) ^ minint
    h_r = (gp[3][:, 129:130] * 256.0 + gp[3][:, 128:129]).astype(jnp.int32)
    base = h_r * w + gp[3][:, 130:131].astype(jnp.int32) * 128
    cand_idx = base + jax.lax.broadcasted_iota(jnp.int32, (_S, 128), 1)
    return cand_keys, cand_idx


def _dilate(m):
    h, w = m.shape
    f = m
    for d in (1, 2, 3):
        up = jnp.concatenate([m[d:, :], jnp.zeros((d, w), jnp.float32)], axis=0)
        dnn = jnp.concatenate([jnp.zeros((d, w), jnp.float32), m[: h - d, :]], axis=0)
        f = jnp.maximum(f, jnp.maximum(up, dnn))
    g = f
    for d in (1, 2, 3):
        lf = jnp.concatenate([f[:, d:], jnp.zeros((h, d), jnp.float32)], axis=1)
        rt = jnp.concatenate([jnp.zeros((h, d), jnp.float32), f[:, : w - d]], axis=1)
        g = jnp.maximum(g, jnp.maximum(lf, rt))
    return g


def _body(rv_ref, ri_ref, rf_ref, out_ref):
    pid = pl.program_id(0)
    h, w = rv_ref.shape[1], rv_ref.shape[2]
    row = jax.lax.broadcasted_iota(jnp.int32, (h, w), 0)
    col = jax.lax.broadcasted_iota(jnp.int32, (h, w), 1)
    idx = row * w + col
    tril = (
        jax.lax.broadcasted_iota(jnp.int32, (h, h), 0)
        > jax.lax.broadcasted_iota(jnp.int32, (h, h), 1)
    ).astype(jnp.float32)
    cid = (
        jax.lax.broadcasted_iota(jnp.int32, (h, _NJ), 0) * _NJ
        + jax.lax.broadcasted_iota(jnp.int32, (h, _NJ), 1)
    )

    xs = [r[0] for r in (rv_ref, ri_ref, rf_ref)]
    keys = [_monotone_key(x) for x in xs]
    ckeys = [_chunk_maxes(kk) for kk in keys]

    cts = _kth3(ckeys, (_S, _S, _S))
    ctie = [ck == t for ck, t in zip(ckeys, cts)]
    crs = [
        jnp.int32(_S) - _csum(ck > t) for ck, t in zip(ckeys, cts)
    ]
    ccuts = _cut3(ctie, (cid, cid, cid), crs, _CID_BITS)
    selcs = [
        ((ck > t) | (ti & (cid <= cu))).astype(jnp.float32)
        for ck, t, ti, cu in zip(ckeys, cts, ctie, ccuts)
    ]

    cands = [
        _compact(kk, sc, _positions(sc, tril)) for kk, sc in zip(keys, selcs)
    ]
    ckq = [ckv for ckv, _ in cands]
    cix = [civ for _, civ in cands]
    kks = (_TOPK, _TOPK, 2 * _TOPK)
    tss = _kth3(ckq, kks)
    ties = [cq == t for cq, t in zip(ckq, tss)]
    rrs = [
        jnp.int32(k) - _csum(cq > t) for cq, t, k in zip(ckq, tss, kks)
    ]
    cuts = _cut3(ties, cix, rrs, _IDX_BITS)

    masks = [
        ((kk > t) | ((kk == t) & (idx <= cu))).astype(jnp.float32)
        for kk, t, cu in zip(keys, tss, cuts)
    ]

    src = jnp.maximum(masks[0], masks[1])
    dil = _dilate(masks[2])
    cover = jnp.sum(src * dil)
    denom = jnp.maximum(jnp.sum(src), 1.0)
    contrib = (1.0 - cover / denom) * jnp.float32(1.0 / _B)

    @pl.when(pid == 0)
    def _():
        out_ref[...] = jnp.zeros_like(out_ref)

    out_ref[...] += contrib


def kernel(rv, ri, rf):
    b, c, h, w = rv.shape
    rv3 = rv.reshape(b, c * h, w)
    ri3 = ri.reshape(b, c * h, w)
    rf3 = rf.reshape(b, c * h, w)
    spec = pl.BlockSpec((1, c * h, w), lambda i: (i, 0, 0))
    out = pl.pallas_call(
        _body,
        grid=(b,),
        in_specs=[spec, spec, spec],
        out_specs=pl.BlockSpec((1, 128), lambda i: (0, 0)),
        out_shape=jax.ShapeDtypeStruct((1, 128), jnp.float32),
    )(rv3, ri3, rf3)
    return out[0, 0]


# speedup vs baseline: 1.3768x; 1.2014x over previous
import jax
import jax.numpy as jnp
from jax.experimental import pallas as pl

_B, _C, _H, _W = 16, 1, 512, 512
_TOPK = 180
_TOL = 3
_IDX_BITS = 18
_NJ = 4
_S = 384
_CID_BITS = 12
_MININT = -2147483648


def _monotone_key(x):
    b = jax.lax.bitcast_convert_type(x, jnp.int32)
    return jnp.where(b < 0, b ^ jnp.int32(0x7FFFFFFF), b)


def _csum(pred):
    return jnp.sum(pred.astype(jnp.int32), keepdims=True)


def _kth3(arrs, ks):
    minint = jnp.int32(_MININT)

    def body(i, ubs):
        shift = 30 - 2 * i
        out = []
        for u, a, k in zip(ubs, arrs, ks):
            p1 = u | jnp.left_shift(jnp.int32(1), shift)
            p2 = u | jnp.left_shift(jnp.int32(2), shift)
            p3 = u | jnp.left_shift(jnp.int32(3), shift)
            c1 = _csum(a >= (p1 ^ minint))
            c2 = _csum(a >= (p2 ^ minint))
            c3 = _csum(a >= (p3 ^ minint))
            nu = jnp.where(c3 >= k, p3, jnp.where(c2 >= k, p2, jnp.where(c1 >= k, p1, u)))
            out.append(nu)
        return tuple(out)

    init = (jnp.zeros((1, 1), jnp.int32),) * 3
    ubs = jax.lax.fori_loop(0, 16, body, init)
    return tuple(u ^ minint for u in ubs)


def _cut1(tie, idv, r, nbits):
    def fast(_):
        return jnp.min(
            jnp.where(tie, idv, jnp.int32(0x7FFFFFFF)), keepdims=True
        ).reshape(1, 1)

    def slow(_):
        def body(i, u):
            shift = (nbits - 2) - 2 * i
            s1 = u | jnp.left_shift(jnp.int32(1), shift)
            s2 = u | jnp.left_shift(jnp.int32(2), shift)
            s3 = u | jnp.left_shift(jnp.int32(3), shift)
            c1 = _csum(tie & (idv < s1))
            c2 = _csum(tie & (idv < s2))
            c3 = _csum(tie & (idv < s3))
            return jnp.where(
                c3 < r, s3, jnp.where(c2 < r, s2, jnp.where(c1 < r, s1, u))
            )

        return jax.lax.fori_loop(0, nbits // 2, body, jnp.zeros((1, 1), jnp.int32))

    return jax.lax.cond(r.reshape(()) == 1, fast, slow, 0)


def _cut3(ties, ids, rs, nbits):
    return tuple(
        _cut1(tie, idv, r, nbits) for tie, idv, r in zip(ties, ids, rs)
    )


def _chunk_maxes(keys):
    cms = [
        jnp.max(keys[:, j * 128 : (j + 1) * 128], axis=1, keepdims=True)
        for j in range(_NJ)
    ]
    return jnp.concatenate(cms, axis=1)


def _positions(selc, tril):
    s = selc
    rowtot = s[:, 0:1] + s[:, 1:2] + s[:, 2:3] + s[:, 3:4]
    cumex = jax.lax.dot_general(
        tril, rowtot, (((1,), (0,)), ((), ()))
    )
    p0 = cumex
    p1 = p0 + s[:, 0:1]
    p2 = p1 + s[:, 1:2]
    p3 = p2 + s[:, 2:3]
    return jnp.concatenate([p0, p1, p2, p3], axis=1)


def _compact(keys, selc, pos):
    minint = jnp.int32(_MININT)
    h, w = keys.shape
    nc = h * _NJ
    ubk = keys ^ minint
    lane_r = jax.lax.broadcasted_iota(jnp.int32, (nc, _S), 1)
    hcol = jax.lax.broadcasted_iota(jnp.int32, (h, 1), 0)
    h_lo = (hcol & 255).astype(jnp.float32)
    h_hi = jax.lax.shift_right_logical(hcol, 8).astype(jnp.float32)

    pos_cat = jnp.concatenate(
        [pos[:, j : j + 1] for j in range(_NJ)], axis=0
    ).astype(jnp.int32)
    sel_cat = jnp.concatenate([selc[:, j : j + 1] for j in range(_NJ)], axis=0)
    q = ((lane_r == pos_cat) & (sel_cat > 0.5)).astype(jnp.float32)

    dn = (((0,), (0,)), ((), ()))
    gp = []
    for p in range(4):
        plane = jnp.concatenate(
            [
                (
                    jax.lax.shift_right_logical(
                        ubk[:, j * 128 : (j + 1) * 128], 8 * (3 - p)
                    )
                    & 255
                ).astype(jnp.float32)
                for j in range(_NJ)
            ],
            axis=0,
        )
        if p == 3:
            plane = jnp.concatenate(
                [
                    plane,
                    jnp.concatenate([h_lo for _ in range(_NJ)], axis=0),
                    jnp.concatenate([h_hi for _ in range(_NJ)], axis=0),
                    jnp.concatenate(
                        [jnp.full((h, 1), float(j), jnp.float32) for j in range(_NJ)],
                        axis=0,
                    ),
                ],
                axis=1,
            )
        gp.append(jax.lax.dot_general(q, plane, dn))

    ip = [g.astype(jnp.int32) for g in (gp[0], gp[1], gp[2], gp[3][:, :128])]
    cand_keys = (
        jnp.left_shift(ip[0], 24)
        | jnp.left_shift(ip[1], 16)
        | jnp.left_shift(ip[2], 8)
        | ip[3]
    ) ^ minint
    h_r = (gp[3][:, 129:130] * 256.0 + gp[3][:, 128:129]).astype(jnp.int32)
    base = h_r * w + gp[3][:, 130:131].astype(jnp.int32) * 128
    cand_idx = base + jax.lax.broadcasted_iota(jnp.int32, (_S, 128), 1)
    return cand_keys, cand_idx


def _dilate(m):
    h, w = m.shape
    f = m
    for d in (1, 2, 3):
        up = jnp.concatenate([m[d:, :], jnp.zeros((d, w), jnp.float32)], axis=0)
        dnn = jnp.concatenate([jnp.zeros((d, w), jnp.float32), m[: h - d, :]], axis=0)
        f = jnp.maximum(f, jnp.maximum(up, dnn))
    g = f
    for d in (1, 2, 3):
        lf = jnp.concatenate([f[:, d:], jnp.zeros((h, d), jnp.float32)], axis=1)
        rt = jnp.concatenate([jnp.zeros((h, d), jnp.float32), f[:, : w - d]], axis=1)
        g = jnp.maximum(g, jnp.maximum(lf, rt))
    return g


def _body(rv_ref, ri_ref, rf_ref, out_ref):
    pid = pl.program_id(0)
    h, w = rv_ref.shape[1], rv_ref.shape[2]
    row = jax.lax.broadcasted_iota(jnp.int32, (h, w), 0)
    col = jax.lax.broadcasted_iota(jnp.int32, (h, w), 1)
    idx = row * w + col
    tril = (
        jax.lax.broadcasted_iota(jnp.int32, (h, h), 0)
        > jax.lax.broadcasted_iota(jnp.int32, (h, h), 1)
    ).astype(jnp.float32)
    cid = (
        jax.lax.broadcasted_iota(jnp.int32, (h, _NJ), 0) * _NJ
        + jax.lax.broadcasted_iota(jnp.int32, (h, _NJ), 1)
    )
    xs = [r[0] for r in (rv_ref, ri_ref, rf_ref)]
    keys = [_monotone_key(x) for x in xs]
    ckeys = [_chunk_maxes(kk) for kk in keys]

    cts = _kth3(ckeys, (_S, _S, _S))
    ctie = [ck == t for ck, t in zip(ckeys, cts)]
    crs = [
        jnp.int32(_S) - _csum(ck > t) for ck, t in zip(ckeys, cts)
    ]
    ccuts = _cut3(ctie, (cid, cid, cid), crs, _CID_BITS)
    selcs = [
        ((ck > t) | (ti & (cid <= cu))).astype(jnp.float32)
        for ck, t, ti, cu in zip(ckeys, cts, ctie, ccuts)
    ]

    cands = [
        _compact(kk, sc, _positions(sc, tril)) for kk, sc in zip(keys, selcs)
    ]
    ckq = [ckv for ckv, _ in cands]
    cix = [civ for _, civ in cands]
    kks = (_TOPK, _TOPK, 2 * _TOPK)
    tss = _kth3(ckq, kks)
    ties = [cq == t for cq, t in zip(ckq, tss)]
    rrs = [
        jnp.int32(k) - _csum(cq > t) for cq, t, k in zip(ckq, tss, kks)
    ]
    cuts = _cut3(ties, cix, rrs, _IDX_BITS)

    masks = [
        ((kk > t) | ((kk == t) & (idx <= cu))).astype(jnp.float32)
        for kk, t, cu in zip(keys, tss, cuts)
    ]

    src = jnp.maximum(masks[0], masks[1])
    dil = _dilate(masks[2])
    cover = jnp.sum(src * dil)
    denom = jnp.maximum(jnp.sum(src), 1.0)
    contrib = (1.0 - cover / denom) * jnp.float32(1.0 / _B)

    @pl.when(pid == 0)
    def _():
        out_ref[...] = jnp.zeros_like(out_ref)

    out_ref[...] += contrib


def kernel(rv, ri, rf):
    b, c, h, w = rv.shape
    rv3 = rv.reshape(b, c * h, w)
    ri3 = ri.reshape(b, c * h, w)
    rf3 = rf.reshape(b, c * h, w)
    spec = pl.BlockSpec((1, c * h, w), lambda i: (i, 0, 0))
    out = pl.pallas_call(
        _body,
        grid=(b,),
        in_specs=[spec, spec, spec],
        out_specs=pl.BlockSpec((1, 128), lambda i: (0, 0)),
        out_shape=jax.ShapeDtypeStruct((1, 128), jnp.float32),
    )(rv3, ri3, rf3)
    return out[0, 0]


# speedup vs baseline: 1.3876x; 1.0078x over previous
import jax
import jax.numpy as jnp
from jax.experimental import pallas as pl

_B, _C, _H, _W = 16, 1, 512, 512
_TOPK = 180
_TOL = 3
_IDX_BITS = 18
_NJ = 4
_S = 384
_CID_BITS = 12
_MININT = -2147483648


def _monotone_key(x):
    b = jax.lax.bitcast_convert_type(x, jnp.int32)
    return jnp.where(b < 0, b ^ jnp.int32(0x7FFFFFFF), b)


def _csum(pred):
    return jnp.sum(pred.astype(jnp.int32), keepdims=True)


def _kth3(arrs, ks):
    minint = jnp.int32(_MININT)

    def body(i, ubs):
        shift = 30 - 2 * i
        out = []
        for u, a, k in zip(ubs, arrs, ks):
            p1 = u | jnp.left_shift(jnp.int32(1), shift)
            p2 = u | jnp.left_shift(jnp.int32(2), shift)
            p3 = u | jnp.left_shift(jnp.int32(3), shift)
            c1 = _csum(a >= (p1 ^ minint))
            c2 = _csum(a >= (p2 ^ minint))
            c3 = _csum(a >= (p3 ^ minint))
            nu = jnp.where(c3 >= k, p3, jnp.where(c2 >= k, p2, jnp.where(c1 >= k, p1, u)))
            out.append(nu)
        return tuple(out)

    init = (jnp.zeros((1, 1), jnp.int32),) * 3
    ubs = jax.lax.fori_loop(0, 16, body, init)
    return tuple(u ^ minint for u in ubs)


def _cut1(tie, idv, r, nbits):
    def fast(_):
        return jnp.min(
            jnp.where(tie, idv, jnp.int32(0x7FFFFFFF)), keepdims=True
        ).reshape(1, 1)

    def slow(_):
        def body(i, u):
            shift = (nbits - 2) - 2 * i
            s1 = u | jnp.left_shift(jnp.int32(1), shift)
            s2 = u | jnp.left_shift(jnp.int32(2), shift)
            s3 = u | jnp.left_shift(jnp.int32(3), shift)
            c1 = _csum(tie & (idv < s1))
            c2 = _csum(tie & (idv < s2))
            c3 = _csum(tie & (idv < s3))
            return jnp.where(
                c3 < r, s3, jnp.where(c2 < r, s2, jnp.where(c1 < r, s1, u))
            )

        return jax.lax.fori_loop(0, nbits // 2, body, jnp.zeros((1, 1), jnp.int32))

    return jax.lax.cond(r.reshape(()) == 1, fast, slow, 0)


def _cut3(ties, ids, rs, nbits):
    return tuple(
        _cut1(tie, idv, r, nbits) for tie, idv, r in zip(ties, ids, rs)
    )


def _chunk_maxes(keys):
    cms = [
        jnp.max(keys[:, j * 128 : (j + 1) * 128], axis=1, keepdims=True)
        for j in range(_NJ)
    ]
    return jnp.concatenate(cms, axis=1)


def _positions(selc, tril):
    s = selc
    rowtot = s[:, 0:1] + s[:, 1:2] + s[:, 2:3] + s[:, 3:4]
    cumex = jax.lax.dot_general(
        tril, rowtot, (((1,), (0,)), ((), ()))
    )
    p0 = cumex
    p1 = p0 + s[:, 0:1]
    p2 = p1 + s[:, 1:2]
    p3 = p2 + s[:, 2:3]
    return jnp.concatenate([p0, p1, p2, p3], axis=1)


def _compact(keys, selc, pos):
    minint = jnp.int32(_MININT)
    h, w = keys.shape
    nc = h * _NJ
    ubk = keys ^ minint
    lane_r = jax.lax.broadcasted_iota(jnp.int32, (nc, _S), 1)
    hcol = jax.lax.broadcasted_iota(jnp.int32, (h, 1), 0)
    h_lo = (hcol & 255).astype(jnp.float32)
    h_hi = jax.lax.shift_right_logical(hcol, 8).astype(jnp.float32)

    pos_v = jnp.where(selc > 0.5, pos, -1.0)
    pos_cat = jnp.concatenate(
        [pos_v[:, j : j + 1] for j in range(_NJ)], axis=0
    ).astype(jnp.int32)
    q = (lane_r == pos_cat).astype(jnp.float32)

    dn = (((0,), (0,)), ((), ()))
    gp = []
    for p in range(4):
        plane = jnp.concatenate(
            [
                (
                    jax.lax.shift_right_logical(
                        ubk[:, j * 128 : (j + 1) * 128], 8 * (3 - p)
                    )
                    & 255
                ).astype(jnp.float32)
                for j in range(_NJ)
            ],
            axis=0,
        )
        if p == 3:
            plane = jnp.concatenate(
                [
                    plane,
                    jnp.concatenate([h_lo for _ in range(_NJ)], axis=0),
                    jnp.concatenate([h_hi for _ in range(_NJ)], axis=0),
                    jnp.concatenate(
                        [jnp.full((h, 1), float(j), jnp.float32) for j in range(_NJ)],
                        axis=0,
                    ),
                ],
                axis=1,
            )
        gp.append(jax.lax.dot_general(q, plane, dn))

    ip = [g.astype(jnp.int32) for g in (gp[0], gp[1], gp[2], gp[3][:, :128])]
    cand_keys = (
        jnp.left_shift(ip[0], 24)
        | jnp.left_shift(ip[1], 16)
        | jnp.left_shift(ip[2], 8)
        | ip[3]
    ) ^ minint
    h_r = (gp[3][:, 129:130] * 256.0 + gp[3][:, 128:129]).astype(jnp.int32)
    base = h_r * w + gp[3][:, 130:131].astype(jnp.int32) * 128
    cand_idx = base + jax.lax.broadcasted_iota(jnp.int32, (_S, 128), 1)
    return cand_keys, cand_idx


def _dilate(m):
    h, w = m.shape
    f = m
    for d in (1, 2, 3):
        up = jnp.concatenate([m[d:, :], jnp.zeros((d, w), jnp.float32)], axis=0)
        dnn = jnp.concatenate([jnp.zeros((d, w), jnp.float32), m[: h - d, :]], axis=0)
        f = jnp.maximum(f, jnp.maximum(up, dnn))
    g = f
    for d in (1, 2, 3):
        lf = jnp.concatenate([f[:, d:], jnp.zeros((h, d), jnp.float32)], axis=1)
        rt = jnp.concatenate([jnp.zeros((h, d), jnp.float32), f[:, : w - d]], axis=1)
        g = jnp.maximum(g, jnp.maximum(lf, rt))
    return g


def _body(rv_ref, ri_ref, rf_ref, out_ref):
    pid = pl.program_id(0)
    h, w = rv_ref.shape[1], rv_ref.shape[2]
    row = jax.lax.broadcasted_iota(jnp.int32, (h, w), 0)
    col = jax.lax.broadcasted_iota(jnp.int32, (h, w), 1)
    idx = row * w + col
    tril = (row > col).astype(jnp.float32)
    cid = (
        jax.lax.broadcasted_iota(jnp.int32, (h, _NJ), 0) * _NJ
        + jax.lax.broadcasted_iota(jnp.int32, (h, _NJ), 1)
    )
    xs = [r[0] for r in (rv_ref, ri_ref, rf_ref)]
    keys = [_monotone_key(x) for x in xs]
    ckeys = [_chunk_maxes(kk) for kk in keys]

    cts = _kth3(ckeys, (_S, _S, _S))
    ctie = [ck == t for ck, t in zip(ckeys, cts)]
    crs = [
        jnp.int32(_S) - _csum(ck > t) for ck, t in zip(ckeys, cts)
    ]
    ccuts = _cut3(ctie, (cid, cid, cid), crs, _CID_BITS)
    selcs = [
        ((ck > t) | (ti & (cid <= cu))).astype(jnp.float32)
        for ck, t, ti, cu in zip(ckeys, cts, ctie, ccuts)
    ]

    cands = [
        _compact(kk, sc, _positions(sc, tril)) for kk, sc in zip(keys, selcs)
    ]
    ckq = [ckv for ckv, _ in cands]
    cix = [civ for _, civ in cands]
    kks = (_TOPK, _TOPK, 2 * _TOPK)
    tss = _kth3(ckq, kks)
    ties = [cq == t for cq, t in zip(ckq, tss)]
    rrs = [
        jnp.int32(k) - _csum(cq > t) for cq, t, k in zip(ckq, tss, kks)
    ]
    cuts = _cut3(ties, cix, rrs, _IDX_BITS)

    masks = [
        ((kk > t) | ((kk == t) & (idx <= cu))).astype(jnp.float32)
        for kk, t, cu in zip(keys, tss, cuts)
    ]

    src = jnp.maximum(masks[0], masks[1])
    dil = _dilate(masks[2])
    cover = jnp.sum(src * dil)
    denom = jnp.maximum(jnp.sum(src), 1.0)
    contrib = (1.0 - cover / denom) * jnp.float32(1.0 / _B)

    @pl.when(pid == 0)
    def _():
        out_ref[...] = jnp.zeros_like(out_ref)

    out_ref[...] += contrib


def kernel(rv, ri, rf):
    b, c, h, w = rv.shape
    rv3 = rv.reshape(b, c * h, w)
    ri3 = ri.reshape(b, c * h, w)
    rf3 = rf.reshape(b, c * h, w)
    spec = pl.BlockSpec((1, c * h, w), lambda i: (i, 0, 0))
    out = pl.pallas_call(
        _body,
        grid=(b,),
        in_specs=[spec, spec, spec],
        out_specs=pl.BlockSpec((1, 128), lambda i: (0, 0)),
        out_shape=jax.ShapeDtypeStruct((1, 128), jnp.float32),
    )(rv3, ri3, rf3)
    return out[0, 0]


# speedup vs baseline: 1.7667x; 1.2732x over previous
import jax
import jax.numpy as jnp
from jax.experimental import pallas as pl

_B, _C, _H, _W = 16, 1, 512, 512
_TOPK = 180
_TOL = 3
_IDX_BITS = 18
_NJ = 4
_S = 384
_CID_BITS = 12
_MININT = -2147483648


def _monotone_key(x):
    b = jax.lax.bitcast_convert_type(x, jnp.int32)
    return jnp.where(b < 0, b ^ jnp.int32(0x7FFFFFFF), b)


def _csum(pred):
    return jnp.sum(pred.astype(jnp.int32), keepdims=True)


def _kth3(arrs, ks):
    minint = jnp.int32(_MININT)

    def body(i, ubs):
        shift = 30 - 2 * i
        out = []
        for u, a, k in zip(ubs, arrs, ks):
            p1 = u | jnp.left_shift(jnp.int32(1), shift)
            p2 = u | jnp.left_shift(jnp.int32(2), shift)
            p3 = u | jnp.left_shift(jnp.int32(3), shift)
            c1 = _csum(a >= (p1 ^ minint))
            c2 = _csum(a >= (p2 ^ minint))
            c3 = _csum(a >= (p3 ^ minint))
            nu = jnp.where(c3 >= k, p3, jnp.where(c2 >= k, p2, jnp.where(c1 >= k, p1, u)))
            out.append(nu)
        return tuple(out)

    init = (jnp.zeros((1, 1), jnp.int32),) * 3
    ubs = jax.lax.fori_loop(0, 16, body, init)
    return tuple(u ^ minint for u in ubs)


def _cut1(tie, idv, r, nbits):
    def fast(_):
        return jnp.min(
            jnp.where(tie, idv, jnp.int32(0x7FFFFFFF)), keepdims=True
        ).reshape(1, 1)

    def slow(_):
        def body(i, u):
            shift = (nbits - 2) - 2 * i
            s1 = u | jnp.left_shift(jnp.int32(1), shift)
            s2 = u | jnp.left_shift(jnp.int32(2), shift)
            s3 = u | jnp.left_shift(jnp.int32(3), shift)
            c1 = _csum(tie & (idv < s1))
            c2 = _csum(tie & (idv < s2))
            c3 = _csum(tie & (idv < s3))
            return jnp.where(
                c3 < r, s3, jnp.where(c2 < r, s2, jnp.where(c1 < r, s1, u))
            )

        return jax.lax.fori_loop(0, nbits // 2, body, jnp.zeros((1, 1), jnp.int32))

    return jax.lax.cond(r.reshape(()) == 1, fast, slow, 0)


def _cut3(ties, ids, rs, nbits):
    return tuple(
        _cut1(tie, idv, r, nbits) for tie, idv, r in zip(ties, ids, rs)
    )


def _chunk_maxes(keys):
    cms = [
        jnp.max(keys[:, j * 128 : (j + 1) * 128], axis=1, keepdims=True)
        for j in range(_NJ)
    ]
    return jnp.concatenate(cms, axis=1)


def _positions(selc, tril):
    s = selc
    rowtot = s[:, 0:1] + s[:, 1:2] + s[:, 2:3] + s[:, 3:4]
    cumex = jax.lax.dot_general(
        tril, rowtot, (((1,), (0,)), ((), ()))
    )
    p0 = cumex
    p1 = p0 + s[:, 0:1]
    p2 = p1 + s[:, 1:2]
    p3 = p2 + s[:, 2:3]
    return jnp.concatenate([p0, p1, p2, p3], axis=1)


def _compact(keys, selc, pos):
    minint = jnp.int32(_MININT)
    h, w = keys.shape
    nc = h * _NJ
    ubk = keys ^ minint
    lane_r = jax.lax.broadcasted_iota(jnp.int32, (nc, _S), 1)
    hcol = jax.lax.broadcasted_iota(jnp.int32, (h, 1), 0)
    h_lo = (hcol & 255).astype(jnp.float32)
    h_hi = jax.lax.shift_right_logical(hcol, 8).astype(jnp.float32)

    pos_v = jnp.where(selc > 0.5, pos, -1.0)
    pos_cat = jnp.concatenate(
        [pos_v[:, j : j + 1] for j in range(_NJ)], axis=0
    ).astype(jnp.int32)
    q = (lane_r == pos_cat).astype(jnp.float32)

    dn = (((0,), (0,)), ((), ()))
    gp = []
    for p in range(4):
        plane = jnp.concatenate(
            [
                (
                    jax.lax.shift_right_logical(
                        ubk[:, j * 128 : (j + 1) * 128], 8 * (3 - p)
                    )
                    & 255
                ).astype(jnp.float32)
                for j in range(_NJ)
            ],
            axis=0,
        )
        if p == 3:
            plane = jnp.concatenate(
                [
                    plane,
                    jnp.concatenate([h_lo for _ in range(_NJ)], axis=0),
                    jnp.concatenate([h_hi for _ in range(_NJ)], axis=0),
                    jnp.concatenate(
                        [jnp.full((h, 1), float(j), jnp.float32) for j in range(_NJ)],
                        axis=0,
                    ),
                ],
                axis=1,
            )
        gp.append(jax.lax.dot_general(q, plane, dn))

    ip = [g.astype(jnp.int32) for g in (gp[0], gp[1], gp[2], gp[3][:, :128])]
    cand_keys = (
        jnp.left_shift(ip[0], 24)
        | jnp.left_shift(ip[1], 16)
        | jnp.left_shift(ip[2], 8)
        | ip[3]
    ) ^ minint
    h_r = (gp[3][:, 129:130] * 256.0 + gp[3][:, 128:129]).astype(jnp.int32)
    base = h_r * w + gp[3][:, 130:131].astype(jnp.int32) * 128
    cand_idx = base + jax.lax.broadcasted_iota(jnp.int32, (_S, 128), 1)
    return cand_keys, cand_idx


def _dilate(m):
    h, w = m.shape
    f = m
    for d in (1, 2, 3):
        up = jnp.concatenate([m[d:, :], jnp.zeros((d, w), jnp.float32)], axis=0)
        dnn = jnp.concatenate([jnp.zeros((d, w), jnp.float32), m[: h - d, :]], axis=0)
        f = jnp.maximum(f, jnp.maximum(up, dnn))
    g = f
    for d in (1, 2, 3):
        lf = jnp.concatenate([f[:, d:], jnp.zeros((h, d), jnp.float32)], axis=1)
        rt = jnp.concatenate([jnp.zeros((h, d), jnp.float32), f[:, : w - d]], axis=1)
        g = jnp.maximum(g, jnp.maximum(lf, rt))
    return g


def _body(rv_ref, ri_ref, rf_ref, out_ref):
    pid = pl.program_id(0)
    h, w = rv_ref.shape[1], rv_ref.shape[2]
    row = jax.lax.broadcasted_iota(jnp.int32, (h, w), 0)
    col = jax.lax.broadcasted_iota(jnp.int32, (h, w), 1)
    idx = row * w + col
    tril = (row > col).astype(jnp.float32)
    cid = (
        jax.lax.broadcasted_iota(jnp.int32, (h, _NJ), 0) * _NJ
        + jax.lax.broadcasted_iota(jnp.int32, (h, _NJ), 1)
    )
    xs = [r[0] for r in (rv_ref, ri_ref, rf_ref)]
    keys = [_monotone_key(x) for x in xs]
    ckeys = [_chunk_maxes(kk) for kk in keys]

    minint = jnp.int32(_MININT)
    ident = (row == col).astype(jnp.float32)
    cats = jnp.concatenate([ck ^ minint for ck in ckeys], axis=1)
    dnt = (((0,), (0,)), ((), ()))
    tps = []
    for p in range(4):
        pf = (jax.lax.shift_right_logical(cats, 8 * (3 - p)) & 255).astype(
            jnp.float32
        )
        tps.append(jax.lax.dot_general(pf, ident, dnt))
    ti = [t.astype(jnp.int32) for t in tps]
    ck_t_all = (
        jnp.left_shift(ti[0], 24)
        | jnp.left_shift(ti[1], 16)
        | jnp.left_shift(ti[2], 8)
        | ti[3]
    ) ^ minint
    ckeys_t = [ck_t_all[_NJ * a : _NJ * a + _NJ, :] for a in range(3)]
    cid_t = (
        jax.lax.broadcasted_iota(jnp.int32, (_NJ, h), 1) * _NJ
        + jax.lax.broadcasted_iota(jnp.int32, (_NJ, h), 0)
    )

    cts = _kth3(ckeys_t, (_S, _S, _S))
    ctie_t = [ck == t for ck, t in zip(ckeys_t, cts)]
    crs = [
        jnp.int32(_S) - _csum(ck > t) for ck, t in zip(ckeys_t, cts)
    ]
    ccuts = _cut3(ctie_t, (cid_t, cid_t, cid_t), crs, _CID_BITS)
    ctie = [ck == t for ck, t in zip(ckeys, cts)]
    selcs = [
        ((ck > t) | (ti & (cid <= cu))).astype(jnp.float32)
        for ck, t, ti, cu in zip(ckeys, cts, ctie, ccuts)
    ]

    cands = [
        _compact(kk, sc, _positions(sc, tril)) for kk, sc in zip(keys, selcs)
    ]
    ckq = [ckv for ckv, _ in cands]
    cix = [civ for _, civ in cands]
    kks = (_TOPK, _TOPK, 2 * _TOPK)
    tss = _kth3(ckq, kks)
    ties = [cq == t for cq, t in zip(ckq, tss)]
    rrs = [
        jnp.int32(k) - _csum(cq > t) for cq, t, k in zip(ckq, tss, kks)
    ]
    cuts = _cut3(ties, cix, rrs, _IDX_BITS)

    masks = [
        ((kk > t) | ((kk == t) & (idx <= cu))).astype(jnp.float32)
        for kk, t, cu in zip(keys, tss, cuts)
    ]

    src = jnp.maximum(masks[0], masks[1])
    dil = _dilate(masks[2])
    cover = jnp.sum(src * dil)
    denom = jnp.maximum(jnp.sum(src), 1.0)
    contrib = (1.0 - cover / denom) * jnp.float32(1.0 / _B)

    @pl.when(pid == 0)
    def _():
        out_ref[...] = jnp.zeros_like(out_ref)

    out_ref[...] += contrib


def kernel(rv, ri, rf):
    b, c, h, w = rv.shape
    rv3 = rv.reshape(b, c * h, w)
    ri3 = ri.reshape(b, c * h, w)
    rf3 = rf.reshape(b, c * h, w)
    spec = pl.BlockSpec((1, c * h, w), lambda i: (i, 0, 0))
    out = pl.pallas_call(
        _body,
        grid=(b,),
        in_specs=[spec, spec, spec],
        out_specs=pl.BlockSpec((1, 128), lambda i: (0, 0)),
        out_shape=jax.ShapeDtypeStruct((1, 128), jnp.float32),
    )(rv3, ri3, rf3)
    return out[0, 0]


# speedup vs baseline: 1.9994x; 1.1317x over previous
import jax
import jax.numpy as jnp
from jax.experimental import pallas as pl

_B, _C, _H, _W = 16, 1, 512, 512
_TOPK = 180
_TOL = 3
_IDX_BITS = 18
_NJ = 4
_S = 384
_CID_BITS = 12
_MININT = -2147483648


def _monotone_key(x):
    b = jax.lax.bitcast_convert_type(x, jnp.int32)
    return jnp.where(b < 0, b ^ jnp.int32(0x7FFFFFFF), b)


def _csum(pred):
    return jnp.sum(pred.astype(jnp.int32), keepdims=True)


def _kth3(arrs, ks):
    minint = jnp.int32(_MININT)

    def body(i, ubs):
        shift = 30 - 2 * i
        out = []
        for u, a, k in zip(ubs, arrs, ks):
            p1 = u | jnp.left_shift(jnp.int32(1), shift)
            p2 = u | jnp.left_shift(jnp.int32(2), shift)
            p3 = u | jnp.left_shift(jnp.int32(3), shift)
            c1 = _csum(a >= (p1 ^ minint))
            c2 = _csum(a >= (p2 ^ minint))
            c3 = _csum(a >= (p3 ^ minint))
            nu = jnp.where(c3 >= k, p3, jnp.where(c2 >= k, p2, jnp.where(c1 >= k, p1, u)))
            out.append(nu)
        return tuple(out)

    init = (jnp.zeros((1, 1), jnp.int32),) * len(arrs)
    ubs = jax.lax.fori_loop(0, 16, body, init)
    return tuple(u ^ minint for u in ubs)


def _cut1(tie, idv, r, nbits):
    def fast(_):
        return jnp.min(
            jnp.where(tie, idv, jnp.int32(0x7FFFFFFF)), keepdims=True
        ).reshape(1, 1)

    def slow(_):
        def body(i, u):
            shift = (nbits - 2) - 2 * i
            s1 = u | jnp.left_shift(jnp.int32(1), shift)
            s2 = u | jnp.left_shift(jnp.int32(2), shift)
            s3 = u | jnp.left_shift(jnp.int32(3), shift)
            c1 = _csum(tie & (idv < s1))
            c2 = _csum(tie & (idv < s2))
            c3 = _csum(tie & (idv < s3))
            return jnp.where(
                c3 < r, s3, jnp.where(c2 < r, s2, jnp.where(c1 < r, s1, u))
            )

        return jax.lax.fori_loop(0, nbits // 2, body, jnp.zeros((1, 1), jnp.int32))

    return jax.lax.cond(r.reshape(()) == 1, fast, slow, 0)


def _cut3(ties, ids, rs, nbits):
    return tuple(
        _cut1(tie, idv, r, nbits) for tie, idv, r in zip(ties, ids, rs)
    )


def _chunk_maxes(keys):
    cms = [
        jnp.max(keys[:, j * 128 : (j + 1) * 128], axis=1, keepdims=True)
        for j in range(_NJ)
    ]
    return jnp.concatenate(cms, axis=1)


def _positions(selc, tril):
    s = selc
    rowtot = s[:, 0:1] + s[:, 1:2] + s[:, 2:3] + s[:, 3:4]
    cumex = jax.lax.dot_general(
        tril, rowtot, (((1,), (0,)), ((), ()))
    )
    p0 = cumex
    p1 = p0 + s[:, 0:1]
    p2 = p1 + s[:, 1:2]
    p3 = p2 + s[:, 2:3]
    return jnp.concatenate([p0, p1, p2, p3], axis=1)


def _compact(keys, selc, pos):
    minint = jnp.int32(_MININT)
    h, w = keys.shape
    nc = h * _NJ
    ubk = keys ^ minint
    lane_r = jax.lax.broadcasted_iota(jnp.int32, (nc, _S), 1)
    hcol = jax.lax.broadcasted_iota(jnp.int32, (h, 1), 0)
    h_lo = (hcol & 255).astype(jnp.float32)
    h_hi = jax.lax.shift_right_logical(hcol, 8).astype(jnp.float32)

    pos_v = jnp.where(selc > 0.5, pos, -1.0)
    pos_cat = jnp.concatenate(
        [pos_v[:, j : j + 1] for j in range(_NJ)], axis=0
    ).astype(jnp.int32)
    q = (lane_r == pos_cat).astype(jnp.float32)

    dn = (((0,), (0,)), ((), ()))
    gp = []
    for p in range(4):
        plane = jnp.concatenate(
            [
                (
                    jax.lax.shift_right_logical(
                        ubk[:, j * 128 : (j + 1) * 128], 8 * (3 - p)
                    )
                    & 255
                ).astype(jnp.float32)
                for j in range(_NJ)
            ],
            axis=0,
        )
        if p == 3:
            plane = jnp.concatenate(
                [
                    plane,
                    jnp.concatenate([h_lo for _ in range(_NJ)], axis=0),
                    jnp.concatenate([h_hi for _ in range(_NJ)], axis=0),
                    jnp.concatenate(
                        [jnp.full((h, 1), float(j), jnp.float32) for j in range(_NJ)],
                        axis=0,
                    ),
                ],
                axis=1,
            )
        gp.append(jax.lax.dot_general(q, plane, dn))

    ip = [g.astype(jnp.int32) for g in (gp[0], gp[1], gp[2], gp[3][:, :128])]
    cand_keys = (
        jnp.left_shift(ip[0], 24)
        | jnp.left_shift(ip[1], 16)
        | jnp.left_shift(ip[2], 8)
        | ip[3]
    ) ^ minint
    h_r = (gp[3][:, 129:130] * 256.0 + gp[3][:, 128:129]).astype(jnp.int32)
    base = h_r * w + gp[3][:, 130:131].astype(jnp.int32) * 128
    cand_idx = base + jax.lax.broadcasted_iota(jnp.int32, (_S, 128), 1)
    return cand_keys, cand_idx


def _dilate(m):
    h, w = m.shape
    f = m
    for d in (1, 2, 3):
        up = jnp.concatenate([m[d:, :], jnp.zeros((d, w), jnp.float32)], axis=0)
        dnn = jnp.concatenate([jnp.zeros((d, w), jnp.float32), m[: h - d, :]], axis=0)
        f = jnp.maximum(f, jnp.maximum(up, dnn))
    g = f
    for d in (1, 2, 3):
        lf = jnp.concatenate([f[:, d:], jnp.zeros((h, d), jnp.float32)], axis=1)
        rt = jnp.concatenate([jnp.zeros((h, d), jnp.float32), f[:, : w - d]], axis=1)
        g = jnp.maximum(g, jnp.maximum(lf, rt))
    return g


def _body(rv_ref, ri_ref, rf_ref, out_ref):
    pid = pl.program_id(0)
    h, w = rv_ref.shape[1], rv_ref.shape[2]
    row = jax.lax.broadcasted_iota(jnp.int32, (h, w), 0)
    col = jax.lax.broadcasted_iota(jnp.int32, (h, w), 1)
    idx = row * w + col
    tril = (row > col).astype(jnp.float32)
    cid = (
        jax.lax.broadcasted_iota(jnp.int32, (h, _NJ), 0) * _NJ
        + jax.lax.broadcasted_iota(jnp.int32, (h, _NJ), 1)
    )
    xs = [
        r[b] for b in (0, 1) for r in (rv_ref, ri_ref, rf_ref)
    ]
    na = len(xs)
    kks = (_TOPK, _TOPK, 2 * _TOPK) * 2
    keys = [_monotone_key(x) for x in xs]
    ckeys = [_chunk_maxes(kk) for kk in keys]

    minint = jnp.int32(_MININT)
    ident = (row == col).astype(jnp.float32)
    cats = jnp.concatenate([ck ^ minint for ck in ckeys], axis=1)
    dnt = (((0,), (0,)), ((), ()))
    tps = []
    for p in range(4):
        pf = (jax.lax.shift_right_logical(cats, 8 * (3 - p)) & 255).astype(
            jnp.float32
        )
        tps.append(jax.lax.dot_general(pf, ident, dnt))
    ti = [t.astype(jnp.int32) for t in tps]
    ck_t_all = (
        jnp.left_shift(ti[0], 24)
        | jnp.left_shift(ti[1], 16)
        | jnp.left_shift(ti[2], 8)
        | ti[3]
    ) ^ minint
    ckeys_t = [ck_t_all[_NJ * a : _NJ * a + _NJ, :] for a in range(na)]
    cid_t = (
        jax.lax.broadcasted_iota(jnp.int32, (_NJ, h), 1) * _NJ
        + jax.lax.broadcasted_iota(jnp.int32, (_NJ, h), 0)
    )

    cts = _kth3(ckeys_t, (_S,) * na)
    ctie_t = [ck == t for ck, t in zip(ckeys_t, cts)]
    crs = [
        jnp.int32(_S) - _csum(ck > t) for ck, t in zip(ckeys_t, cts)
    ]
    ccuts = _cut3(ctie_t, (cid_t,) * na, crs, _CID_BITS)
    ctie = [ck == t for ck, t in zip(ckeys, cts)]
    selcs = [
        ((ck > t) | (ti & (cid <= cu))).astype(jnp.float32)
        for ck, t, ti, cu in zip(ckeys, cts, ctie, ccuts)
    ]

    cands = [
        _compact(kk, sc, _positions(sc, tril)) for kk, sc in zip(keys, selcs)
    ]
    ckq = [ckv for ckv, _ in cands]
    cix = [civ for _, civ in cands]
    tss = _kth3(ckq, kks)
    ties = [cq == t for cq, t in zip(ckq, tss)]
    rrs = [
        jnp.int32(k) - _csum(cq > t) for cq, t, k in zip(ckq, tss, kks)
    ]
    cuts = _cut3(ties, cix, rrs, _IDX_BITS)

    masks = [
        ((kk > t) | ((kk == t) & (idx <= cu))).astype(jnp.float32)
        for kk, t, cu in zip(keys, tss, cuts)
    ]

    contrib = jnp.float32(0.0)
    for b in (0, 1):
        src = jnp.maximum(masks[3 * b + 0], masks[3 * b + 1])
        dil = _dilate(masks[3 * b + 2])
        cover = jnp.sum(src * dil)
        denom = jnp.maximum(jnp.sum(src), 1.0)
        contrib = contrib + (1.0 - cover / denom) * jnp.float32(1.0 / _B)

    @pl.when(pid == 0)
    def _():
        out_ref[...] = jnp.zeros_like(out_ref)

    out_ref[...] += contrib


def kernel(rv, ri, rf):
    b, c, h, w = rv.shape
    rv3 = rv.reshape(b, c * h, w)
    ri3 = ri.reshape(b, c * h, w)
    rf3 = rf.reshape(b, c * h, w)
    spec = pl.BlockSpec((2, c * h, w), lambda i: (i, 0, 0))
    out = pl.pallas_call(
        _body,
        grid=(b // 2,),
        in_specs=[spec, spec, spec],
        out_specs=pl.BlockSpec((1, 128), lambda i: (0, 0)),
        out_shape=jax.ShapeDtypeStruct((1, 128), jnp.float32),
    )(rv3, ri3, rf3)
    return out[0, 0]
